# Initial kernel scaffold; baseline (speedup 1.0000x reference)
import functools

import jax
import jax.numpy as jnp
from jax import lax
from jax.experimental import pallas as pl
from jax.experimental.pallas import tpu as pltpu
from jax.experimental.pallas import tpu_sc as plsc

N = 10000
NP = 10240
E = 320000
NW = 32
CH = 128
NCHUNK = 80
EP = NW * NCHUNK * CH
STRIPE = NP // 16
H = 128
FFN = 256
B = 16
R = 1024
G = NP // R
RK = 512
GK = NP // RK

_mesh = plsc.VectorSubcoreMesh(core_axis_name="c", subcore_axis_name="s")


def _make_edge_kernel():
    @functools.partial(
        pl.kernel, mesh=_mesh,
        out_type=jax.ShapeDtypeStruct((2, NP, H), jnp.float32),
        scratch_types=[
            pltpu.VMEM((NCHUNK, CH), jnp.int32),
            pltpu.VMEM((NCHUNK, CH), jnp.int32),
            pltpu.VMEM((CH, H), jnp.float32),
            pltpu.VMEM_SHARED((NP, H), jnp.float32),
            pltpu.SemaphoreType.DMA,
        ],
    )
    def k(table, srcp, dstp, out, src_v, dst_v, buf, acc, sem):
        c = lax.axis_index("c")
        s = lax.axis_index("s")
        wid = s * 2 + c
        pltpu.sync_copy(srcp.at[wid], src_v)
        pltpu.sync_copy(dstp.at[wid], dst_v)
        zz = jnp.zeros((16,), jnp.float32)

        def zb(r, _):
            for c8 in range(H // 16):
                buf[r, pl.ds(c8 * 16, 16)] = zz
            return 0

        lax.fori_loop(0, CH, zb, 0)
        base = s * STRIPE
        for t in range(STRIPE // CH):
            pltpu.sync_copy(buf, acc.at[pl.ds(base + t * CH, CH)])
        plsc.subcore_barrier()

        def body(j, _):
            pltpu.async_copy(table.at[src_v.at[j]], buf, sem).wait()
            pltpu.sync_copy(buf, acc.at[dst_v.at[j]], add=True)
            return 0

        lax.fori_loop(0, NCHUNK, body, 0)
        plsc.subcore_barrier()
        pltpu.sync_copy(acc.at[pl.ds(base, STRIPE)],
                        out.at[c, pl.ds(base, STRIPE)])

    return k


def _make_deg_kernel(gather):
    scratch = [
        pltpu.VMEM((NCHUNK, CH), jnp.int32),
        pltpu.VMEM((NCHUNK, CH), jnp.int32),
        pltpu.VMEM((CH,), jnp.float32),
        pltpu.VMEM_SHARED((NP,), jnp.float32),
        pltpu.SemaphoreType.DMA,
    ]

    def body_common(c, s, srcp, dstp, out, src_v, dst_v, buf, acc, sem, vals):
        wid = s * 2 + c
        pltpu.sync_copy(srcp.at[wid], src_v)
        pltpu.sync_copy(dstp.at[wid], dst_v)
        zz = jnp.zeros((16,), jnp.float32)

        def zb(i, _):
            buf[pl.ds(i * 16, 16)] = zz
            return 0

        lax.fori_loop(0, CH // 16, zb, 0)
        base = s * STRIPE
        for t in range(STRIPE // CH):
            pltpu.sync_copy(buf, acc.at[pl.ds(base + t * CH, CH)])
        plsc.subcore_barrier()
        if not gather:
            oo = jnp.ones((16,), jnp.float32)

            def ob(i, _):
                buf[pl.ds(i * 16, 16)] = oo
                return 0

            lax.fori_loop(0, CH // 16, ob, 0)

        def body(j, _):
            if gather:
                pltpu.async_copy(vals.at[src_v.at[j]], buf, sem).wait()
            pltpu.sync_copy(buf, acc.at[dst_v.at[j]], add=True)
            return 0

        lax.fori_loop(0, NCHUNK, body, 0)
        plsc.subcore_barrier()
        pltpu.sync_copy(acc.at[pl.ds(base, STRIPE)],
                        out.at[c, pl.ds(base, STRIPE)])

    if gather:
        @functools.partial(
            pl.kernel, mesh=_mesh,
            out_type=jax.ShapeDtypeStruct((2, NP), jnp.float32),
            scratch_types=scratch,
        )
        def kg(vals, srcp, dstp, out, src_v, dst_v, buf, acc, sem):
            body_common(lax.axis_index("c"), lax.axis_index("s"), srcp, dstp,
                        out, src_v, dst_v, buf, acc, sem, vals)

        return kg

    @functools.partial(
        pl.kernel, mesh=_mesh,
        out_type=jax.ShapeDtypeStruct((2, NP), jnp.float32),
        scratch_types=scratch,
    )
    def k1(srcp, dstp, out, src_v, dst_v, buf, acc, sem):
        body_common(lax.axis_index("c"), lax.axis_index("s"), srcp, dstp,
                    out, src_v, dst_v, buf, acc, sem, None)

    return k1


_edge_kernel = _make_edge_kernel()
_deg_kernel_ones = _make_deg_kernel(gather=False)
_deg_kernel_gather = _make_deg_kernel(gather=True)


def _edge_pass(table, srcP, dstP):
    return _edge_kernel(table, srcP, dstP)


def _deg_pass_ones(srcP, dstP):
    return _deg_kernel_ones(srcP, dstP)


def _deg_pass_gather(vals, srcP, dstP):
    return _deg_kernel_gather(vals, srcP, dstP)


def _ln_in(xb, g, b):
    m = jnp.mean(xb, -1, keepdims=True)
    d = xb - m
    v = jnp.mean(d * d, -1, keepdims=True)
    return d / jnp.sqrt(v + 1e-5) * g + b


def _full(shape):
    return pl.BlockSpec(shape, lambda *_: tuple(0 for _ in shape))


def _rows(width):
    return pl.BlockSpec((R, width), lambda i: (i, 0))


def _pre_body(mode, refs):
    if mode == "emb":
        (x_r, ew_r, eb_r, g1_r, b1_r, wi_r, bi_r, g2_r, b2_r,
         h0_r, u_r, v_r) = refs
    elif mode == "pool":
        (x_r, sc_r, kp_r, g1_r, b1_r, wi_r, bi_r, g2_r, b2_r,
         h0_r, u_r, v_r) = refs
    else:
        (x_r, kp_r, g1_r, b1_r, wi_r, bi_r, g2_r, b2_r, u_r, v_r) = refs
    xb = x_r[...]
    if mode == "emb":
        xb = jnp.dot(xb, ew_r[...],
                     preferred_element_type=jnp.float32) + eb_r[...]
        h0_r[...] = xb
    if mode in ("pool", "masked"):
        kp = kp_r[...]
    if mode == "pool":
        xb = xb * jnp.tanh(sc_r[...]) * kp
        h0_r[...] = xb
    h = _ln_in(xb, g1_r[...], b1_r[...])
    h = jnp.dot(h, wi_r[...], preferred_element_type=jnp.float32) + bi_r[...]
    h = jax.nn.gelu(h)
    u = h[:, :H]
    v = _ln_in(h[:, H:], g2_r[...], b2_r[...])
    if mode in ("pool", "masked"):
        v = v * kp
    u_r[...] = u
    v_r[...] = v


def _pre_call(mode, x, p, extra=None):
    fout = jax.ShapeDtypeStruct((NP, H), jnp.float32)
    body = functools.partial(_pre_body, mode)
    in_specs = [_rows(H)]
    args = [x]
    if mode == "emb":
        in_specs += [_full((H, H)), _full((1, H))]
        args += [extra[0], extra[1].reshape(1, H)]
    if mode == "pool":
        in_specs += [_rows(1), _rows(1)]
        args += [extra[0], extra[1]]
    if mode == "masked":
        in_specs += [_rows(1)]
        args += [extra]
    in_specs += [_full((1, H)), _full((1, H)), _full((H, FFN)),
                 _full((1, FFN)), _full((1, H)), _full((1, H))]
    args += [p["ln1_g"].reshape(1, H), p["ln1_b"].reshape(1, H), p["Win"],
             p["bin"].reshape(1, FFN), p["ln2_g"].reshape(1, H),
             p["ln2_b"].reshape(1, H)]
    n_out = 3 if mode in ("emb", "pool") else 2
    return pl.pallas_call(
        body, grid=(G,),
        in_specs=in_specs,
        out_specs=[_rows(H)] * n_out,
        out_shape=[fout] * n_out,
    )(*args)


def _post_body(refs):
    (x_r, u_r, acc_r, deg_r, ws_r, bs_r, wo_r, bo_r, o_r) = refs
    acc = acc_r[0] + acc_r[1]
    dg = deg_r[0] + deg_r[1]
    agg = acc / jnp.maximum(dg, 1.0)
    gate = jnp.dot(agg, ws_r[...],
                   preferred_element_type=jnp.float32) + bs_r[...]
    o_r[...] = x_r[...] + jnp.dot(
        u_r[...] * gate, wo_r[...],
        preferred_element_type=jnp.float32) + bo_r[...]


def _post_call(xin, u, accP, degP, p):
    return pl.pallas_call(
        lambda *refs: _post_body(refs), grid=(G,),
        in_specs=[
            _rows(H), _rows(H),
            pl.BlockSpec((2, R, H), lambda i: (0, i, 0)),
            pl.BlockSpec((2, R, 1), lambda i: (0, i, 0)),
            _full((H, H)), _full((1, H)), _full((H, H)), _full((1, H)),
        ],
        out_specs=_rows(H),
        out_shape=jax.ShapeDtypeStruct((NP, H), jnp.float32),
    )(xin, u, accP, degP.reshape(2, NP, 1), p["Wsgu"],
      p["bsgu"].reshape(1, H), p["Wout"], p["bout"].reshape(1, H))


def _score_body(refs):
    (x_r, acc_r, w1_r, w2_r, b_r, o_r) = refs
    nb = acc_r[0] + acc_r[1]
    o_r[...] = (jnp.dot(x_r[...], w1_r[...],
                        preferred_element_type=jnp.float32)
                + jnp.dot(nb, w2_r[...],
                          preferred_element_type=jnp.float32) + b_r[...])


def _score_call(x, accP, pp):
    return pl.pallas_call(
        lambda *refs: _score_body(refs), grid=(G,),
        in_specs=[
            _rows(H),
            pl.BlockSpec((2, R, H), lambda i: (0, i, 0)),
            _full((H, 1)), _full((H, 1)), _full((1, 1)),
        ],
        out_specs=_rows(1),
        out_shape=jax.ShapeDtypeStruct((NP, 1), jnp.float32),
    )(x, accP, pp["W1"], pp["W2"], pp["b"].reshape(1, 1))


def _topk_body(sc_r, bc_r, st_r, bt_r, o_r, rank_r, cnt_r):
    i = pl.program_id(0)
    j = pl.program_id(1)

    @pl.when(j == 0)
    def _():
        rank_r[...] = jnp.zeros((RK, 1), jnp.float32)
        cnt_r[...] = jnp.zeros((RK, 1), jnp.float32)

    bi = bc_r[...]
    bj = bt_r[...].reshape(1, RK)
    overlap = jnp.logical_and(jnp.max(bj) >= jnp.min(bi),
                              jnp.min(bj) <= jnp.max(bi))

    @pl.when(overlap)
    def _():
        si = sc_r[...]
        sj = st_r[...].reshape(1, RK)
        same = bi == bj
        ii = lax.broadcasted_iota(jnp.int32, (RK, 1), 0) + i * RK
        jj = lax.broadcasted_iota(jnp.int32, (1, RK), 1) + j * RK
        better = jnp.logical_and(
            same, jnp.logical_or(sj > si,
                                 jnp.logical_and(sj == si, jj < ii)))
        rank_r[...] += jnp.sum(better.astype(jnp.float32), axis=1,
                               keepdims=True)
        cnt_r[...] += jnp.sum(same.astype(jnp.float32), axis=1,
                              keepdims=True)

    @pl.when(j == GK - 1)
    def _():
        k = jnp.maximum(1.0, jnp.ceil(0.5 * cnt_r[...]))
        keep = jnp.logical_and(rank_r[...] < k, bi < B)
        o_r[...] = keep.astype(jnp.float32)


def _topk_call(score, batch_col):
    scoreT = score.reshape(GK, 1, RK)
    batchT = batch_col.reshape(GK, 1, RK)
    return pl.pallas_call(
        _topk_body, grid=(GK, GK),
        in_specs=[
            pl.BlockSpec((RK, 1), lambda i, j: (i, 0)),
            pl.BlockSpec((RK, 1), lambda i, j: (i, 0)),
            pl.BlockSpec((1, 1, RK), lambda i, j: (j, 0, 0)),
            pl.BlockSpec((1, 1, RK), lambda i, j: (j, 0, 0)),
        ],
        out_specs=pl.BlockSpec((RK, 1), lambda i, j: (i, 0)),
        out_shape=jax.ShapeDtypeStruct((NP, 1), jnp.float32),
        scratch_shapes=[pltpu.VMEM((RK, 1), jnp.float32),
                        pltpu.VMEM((RK, 1), jnp.float32)],
    )(score, batch_col, scoreT, batchT)


def _readout_body(x1_r, x2_r, x3_r, x4_r, bb_r, kp_r,
                  w1_r, b1_r, w2_r, b2_r, w3_r, b3_r, o_r,
                  s1, s2, s3, s4, m1, m2, m3, m4, cnt, cntm):
    i = pl.program_id(0)

    @pl.when(i == 0)
    def _():
        for s in (s1, s2, s3, s4):
            s[...] = jnp.zeros((B, H), jnp.float32)
        for m in (m1, m2, m3, m4):
            m[...] = jnp.full((B, H), -jnp.inf, jnp.float32)
        cnt[...] = jnp.zeros((B, 1), jnp.float32)
        cntm[...] = jnp.zeros((B, 1), jnp.float32)

    bb = bb_r[...]
    kp = kp_r[...]
    x1 = x1_r[...]
    x2 = x2_r[...]
    x3 = x3_r[...]
    x4 = x4_r[...]
    oh = (bb == lax.broadcasted_iota(jnp.int32, (1, B), 1)).astype(jnp.float32)
    ohm = oh * kp
    dn = (((0,), (0,)), ((), ()))
    s1[...] += lax.dot_general(oh, x1, dn, preferred_element_type=jnp.float32)
    s2[...] += lax.dot_general(oh, x2, dn, preferred_element_type=jnp.float32)
    s3[...] += lax.dot_general(ohm, x3, dn, preferred_element_type=jnp.float32)
    s4[...] += lax.dot_general(ohm, x4, dn, preferred_element_type=jnp.float32)
    ones_c = jnp.ones((R, 1), jnp.float32)
    cnt[...] += lax.dot_general(oh, ones_c, dn,
                                preferred_element_type=jnp.float32)
    cntm[...] += lax.dot_general(ohm, ones_c, dn,
                                 preferred_element_type=jnp.float32)
    neg = jnp.float32(-jnp.inf)
    kpos = kp > 0.0
    for g in range(B):
        mg = bb == g
        mgm = jnp.logical_and(mg, kpos)
        for m_r, xb, msk in ((m1, x1, mg), (m2, x2, mg),
                             (m3, x3, mgm), (m4, x4, mgm)):
            cur = m_r[pl.ds(g, 1), :]
            new = jnp.max(jnp.where(msk, xb, neg), axis=0, keepdims=True)
            m_r[pl.ds(g, 1), :] = jnp.maximum(cur, new)

    @pl.when(i == G - 1)
    def _():
        c = jnp.maximum(cnt[...], 1.0)
        cm = jnp.maximum(cntm[...], 1.0)
        mx = [jnp.where(m[...] == -jnp.inf, 0.0, m[...])
              for m in (m1, m2, m3, m4)]
        jk0 = jax.nn.gelu(jnp.concatenate(
            [s1[...] / c, s2[...] / c, mx[0], mx[1]], axis=1))
        jk1 = jax.nn.gelu(jnp.concatenate(
            [s3[...] / cm, s4[...] / cm, mx[2], mx[3]], axis=1))
        z = jk0 + jk1
        z = jnp.maximum(jnp.dot(z, w1_r[...],
                                preferred_element_type=jnp.float32)
                        + b1_r[...], 0.0)
        z = jnp.maximum(jnp.dot(z, w2_r[...],
                                preferred_element_type=jnp.float32)
                        + b2_r[...], 0.0)
        o_r[...] = jnp.dot(z, w3_r[...],
                           preferred_element_type=jnp.float32) + b3_r[...]


def _readout_call(x1, x2, x3, x4, batch_col, keep, params):
    nc = 10
    return pl.pallas_call(
        _readout_body, grid=(G,),
        in_specs=[
            _rows(H), _rows(H), _rows(H), _rows(H), _rows(1), _rows(1),
            _full((4 * H, H)), _full((1, H)), _full((H, H)), _full((1, H)),
            _full((H, nc)), _full((1, nc)),
        ],
        out_specs=pl.BlockSpec((B, nc), lambda i: (0, 0)),
        out_shape=jax.ShapeDtypeStruct((B, nc), jnp.float32),
        scratch_shapes=[pltpu.VMEM((B, H), jnp.float32)] * 8
        + [pltpu.VMEM((B, 1), jnp.float32)] * 2,
    )(x1, x2, x3, x4, batch_col, keep,
      params["W1"], params["b1"].reshape(1, H),
      params["W2"], params["b2"].reshape(1, H),
      params["W3"], params["b3"].reshape(1, nc))


def kernel(x, params, edge_index, batch):
    src = edge_index[0]
    dst = edge_index[1]
    pad_n = EP - E
    ar = jnp.arange(pad_n, dtype=jnp.int32)
    pad_src = (ar * 37) % N
    pad_dst = N + (ar % (NP - N))
    srcP = jnp.concatenate([src, pad_src]).reshape(NW, NCHUNK, CH)
    dstP = jnp.concatenate([dst, pad_dst]).reshape(NW, NCHUNK, CH)
    x_p = jnp.pad(x, ((0, NP - N), (0, 0)))
    batch_col = jnp.pad(batch, (0, NP - N),
                        constant_values=B).reshape(NP, 1)

    p0a, p0b = params["block0"]
    p1a, p1b = params["block1"]

    deg0 = _deg_pass_ones(srcP, dstP)

    h0, u1, v1 = _pre_call("emb", x_p, p0a,
                           (params["emb_W"], params["emb_b"]))
    acc1 = _edge_pass(v1, srcP, dstP)
    x1 = _post_call(h0, u1, acc1, deg0, p0a)

    u2, v2 = _pre_call("plain", x1, p0b)
    acc2 = _edge_pass(v2, srcP, dstP)
    x2 = _post_call(x1, u2, acc2, deg0, p0b)

    accp = _edge_pass(x2, srcP, dstP)
    score = _score_call(x2, accp, params["pool"])
    keep = _topk_call(score, batch_col)

    deg2 = _deg_pass_gather(keep.reshape(NP), srcP, dstP)

    xk, u3, v3 = _pre_call("pool", x2, p1a, (score, keep))
    acc3 = _edge_pass(v3, srcP, dstP)
    x3 = _post_call(xk, u3, acc3, deg2, p1a)

    u4, v4 = _pre_call("masked", x3, p1b, keep)
    acc4 = _edge_pass(v4, srcP, dstP)
    x4 = _post_call(x3, u4, acc4, deg2, p1b)

    return _readout_call(x1, x2, x3, x4, batch_col, keep, params)


# speedup vs baseline: 11.6886x; 11.6886x over previous
import functools

import jax
import jax.numpy as jnp
from jax import lax
from jax.experimental import pallas as pl
from jax.experimental.pallas import tpu as pltpu
from jax.experimental.pallas import tpu_sc as plsc

N = 10000
NP = 10240
E = 320000
NW = 32
CH = 128
NCHUNK = 80
EP = NW * NCHUNK * CH
STRIPE = NP // 16
H = 128
FFN = 256
B = 16
R = 1024
G = NP // R
RK = 512
GK = NP // RK


@functools.cache
def _make_edge_kernel():
    _mesh = plsc.VectorSubcoreMesh(core_axis_name="c", subcore_axis_name="s")

    @functools.partial(
        pl.kernel, mesh=_mesh,
        out_type=jax.ShapeDtypeStruct((2, NP, H), jnp.float32),
        scratch_types=[
            pltpu.VMEM((NCHUNK, CH), jnp.int32),
            pltpu.VMEM((NCHUNK, CH), jnp.int32),
            pltpu.VMEM((CH, H), jnp.float32),
            pltpu.VMEM_SHARED((NP, H), jnp.float32),
            pltpu.SemaphoreType.DMA,
        ],
    )
    def k(table, srcp, dstp, out, src_v, dst_v, buf, acc, sem):
        c = lax.axis_index("c")
        s = lax.axis_index("s")
        wid = s * 2 + c
        pltpu.sync_copy(srcp.at[wid], src_v)
        pltpu.sync_copy(dstp.at[wid], dst_v)
        zz = jnp.zeros((16,), jnp.float32)

        def zb(r, _):
            for c8 in range(H // 16):
                buf[r, pl.ds(c8 * 16, 16)] = zz
            return 0

        lax.fori_loop(0, CH, zb, 0)
        base = s * STRIPE
        for t in range(STRIPE // CH):
            pltpu.sync_copy(buf, acc.at[pl.ds(base + t * CH, CH)])
        plsc.subcore_barrier()

        def body(j, _):
            pltpu.async_copy(table.at[src_v.at[j]], buf, sem).wait()
            pltpu.sync_copy(buf, acc.at[dst_v.at[j]], add=True)
            return 0

        lax.fori_loop(0, NCHUNK, body, 0)
        plsc.subcore_barrier()
        pltpu.sync_copy(acc.at[pl.ds(base, STRIPE)],
                        out.at[c, pl.ds(base, STRIPE)])

    return k


@functools.cache
def _make_deg_kernel(gather):
    _mesh = plsc.VectorSubcoreMesh(core_axis_name="c", subcore_axis_name="s")
    scratch = [
        pltpu.VMEM((NCHUNK, CH), jnp.int32),
        pltpu.VMEM((NCHUNK, CH), jnp.int32),
        pltpu.VMEM((CH,), jnp.float32),
        pltpu.VMEM_SHARED((NP,), jnp.float32),
        pltpu.SemaphoreType.DMA,
    ]

    def body_common(c, s, srcp, dstp, out, src_v, dst_v, buf, acc, sem, vals):
        wid = s * 2 + c
        pltpu.sync_copy(srcp.at[wid], src_v)
        pltpu.sync_copy(dstp.at[wid], dst_v)
        zz = jnp.zeros((16,), jnp.float32)

        def zb(i, _):
            buf[pl.ds(i * 16, 16)] = zz
            return 0

        lax.fori_loop(0, CH // 16, zb, 0)
        base = s * STRIPE
        for t in range(STRIPE // CH):
            pltpu.sync_copy(buf, acc.at[pl.ds(base + t * CH, CH)])
        plsc.subcore_barrier()
        if not gather:
            oo = jnp.ones((16,), jnp.float32)

            def ob(i, _):
                buf[pl.ds(i * 16, 16)] = oo
                return 0

            lax.fori_loop(0, CH // 16, ob, 0)

        def body(j, _):
            if gather:
                pltpu.async_copy(vals.at[src_v.at[j]], buf, sem).wait()
            pltpu.sync_copy(buf, acc.at[dst_v.at[j]], add=True)
            return 0

        lax.fori_loop(0, NCHUNK, body, 0)
        plsc.subcore_barrier()
        pltpu.sync_copy(acc.at[pl.ds(base, STRIPE)],
                        out.at[c, pl.ds(base, STRIPE)])

    if gather:
        @functools.partial(
            pl.kernel, mesh=_mesh,
            out_type=jax.ShapeDtypeStruct((2, NP), jnp.float32),
            scratch_types=scratch,
        )
        def kg(vals, srcp, dstp, out, src_v, dst_v, buf, acc, sem):
            body_common(lax.axis_index("c"), lax.axis_index("s"), srcp, dstp,
                        out, src_v, dst_v, buf, acc, sem, vals)

        return kg

    @functools.partial(
        pl.kernel, mesh=_mesh,
        out_type=jax.ShapeDtypeStruct((2, NP), jnp.float32),
        scratch_types=scratch,
    )
    def k1(srcp, dstp, out, src_v, dst_v, buf, acc, sem):
        body_common(lax.axis_index("c"), lax.axis_index("s"), srcp, dstp,
                    out, src_v, dst_v, buf, acc, sem, None)

    return k1


def _edge_pass(table, srcP, dstP):
    return _make_edge_kernel()(table, srcP, dstP)


def _deg_pass_ones(srcP, dstP):
    return _make_deg_kernel(False)(srcP, dstP)


def _deg_pass_gather(vals, srcP, dstP):
    return _make_deg_kernel(True)(vals, srcP, dstP)


def _ln_in(xb, g, b):
    m = jnp.mean(xb, -1, keepdims=True)
    d = xb - m
    v = jnp.mean(d * d, -1, keepdims=True)
    return d / jnp.sqrt(v + 1e-5) * g + b


def _full(shape):
    return pl.BlockSpec(shape, lambda *_: tuple(0 for _ in shape))


def _rows(width):
    return pl.BlockSpec((R, width), lambda i: (i, 0))


def _pre_body(mode, *refs):
    if mode == "emb":
        (x_r, ew_r, eb_r, g1_r, b1_r, wi_r, bi_r, g2_r, b2_r,
         h0_r, u_r, v_r) = refs
    elif mode == "pool":
        (x_r, sc_r, kp_r, g1_r, b1_r, wi_r, bi_r, g2_r, b2_r,
         h0_r, u_r, v_r) = refs
    elif mode == "masked":
        (x_r, kp_r, g1_r, b1_r, wi_r, bi_r, g2_r, b2_r, u_r, v_r) = refs
    else:
        (x_r, g1_r, b1_r, wi_r, bi_r, g2_r, b2_r, u_r, v_r) = refs
    xb = x_r[...]
    if mode == "emb":
        xb = jnp.dot(xb, ew_r[...],
                     preferred_element_type=jnp.float32) + eb_r[...]
        h0_r[...] = xb
    if mode in ("pool", "masked"):
        kp = kp_r[...]
    if mode == "pool":
        xb = xb * jnp.tanh(sc_r[...]) * kp
        h0_r[...] = xb
    h = _ln_in(xb, g1_r[...], b1_r[...])
    h = jnp.dot(h, wi_r[...], preferred_element_type=jnp.float32) + bi_r[...]
    h = jax.nn.gelu(h)
    u = h[:, :H]
    v = _ln_in(h[:, H:], g2_r[...], b2_r[...])
    if mode in ("pool", "masked"):
        v = v * kp
    u_r[...] = u
    v_r[...] = v


def _pre_call(mode, x, p, extra=None):
    fout = jax.ShapeDtypeStruct((NP, H), jnp.float32)
    body = functools.partial(_pre_body, mode)
    in_specs = [_rows(H)]
    args = [x]
    if mode == "emb":
        in_specs += [_full((H, H)), _full((1, H))]
        args += [extra[0], extra[1].reshape(1, H)]
    if mode == "pool":
        in_specs += [_rows(1), _rows(1)]
        args += [extra[0], extra[1]]
    if mode == "masked":
        in_specs += [_rows(1)]
        args += [extra]
    in_specs += [_full((1, H)), _full((1, H)), _full((H, FFN)),
                 _full((1, FFN)), _full((1, H)), _full((1, H))]
    args += [p["ln1_g"].reshape(1, H), p["ln1_b"].reshape(1, H), p["Win"],
             p["bin"].reshape(1, FFN), p["ln2_g"].reshape(1, H),
             p["ln2_b"].reshape(1, H)]
    n_out = 3 if mode in ("emb", "pool") else 2
    return pl.pallas_call(
        body, grid=(G,),
        in_specs=in_specs,
        out_specs=[_rows(H)] * n_out,
        out_shape=[fout] * n_out,
    )(*args)


def _post_body(refs):
    (x_r, u_r, acc_r, deg_r, ws_r, bs_r, wo_r, bo_r, o_r) = refs
    acc = acc_r[0] + acc_r[1]
    dg = deg_r[0] + deg_r[1]
    agg = acc / jnp.maximum(dg, 1.0)
    gate = jnp.dot(agg, ws_r[...],
                   preferred_element_type=jnp.float32) + bs_r[...]
    o_r[...] = x_r[...] + jnp.dot(
        u_r[...] * gate, wo_r[...],
        preferred_element_type=jnp.float32) + bo_r[...]


def _post_call(xin, u, accP, degP, p):
    return pl.pallas_call(
        lambda *refs: _post_body(refs), grid=(G,),
        in_specs=[
            _rows(H), _rows(H),
            pl.BlockSpec((2, R, H), lambda i: (0, i, 0)),
            pl.BlockSpec((2, R, 1), lambda i: (0, i, 0)),
            _full((H, H)), _full((1, H)), _full((H, H)), _full((1, H)),
        ],
        out_specs=_rows(H),
        out_shape=jax.ShapeDtypeStruct((NP, H), jnp.float32),
    )(xin, u, accP, degP.reshape(2, NP, 1), p["Wsgu"],
      p["bsgu"].reshape(1, H), p["Wout"], p["bout"].reshape(1, H))


def _score_body(refs):
    (x_r, acc_r, w1_r, w2_r, b_r, o_r) = refs
    nb = acc_r[0] + acc_r[1]
    o_r[...] = (jnp.dot(x_r[...], w1_r[...],
                        preferred_element_type=jnp.float32)
                + jnp.dot(nb, w2_r[...],
                          preferred_element_type=jnp.float32) + b_r[...])


def _score_call(x, accP, pp):
    return pl.pallas_call(
        lambda *refs: _score_body(refs), grid=(G,),
        in_specs=[
            _rows(H),
            pl.BlockSpec((2, R, H), lambda i: (0, i, 0)),
            _full((H, 1)), _full((H, 1)), _full((1, 1)),
        ],
        out_specs=_rows(1),
        out_shape=jax.ShapeDtypeStruct((NP, 1), jnp.float32),
    )(x, accP, pp["W1"], pp["W2"], pp["b"].reshape(1, 1))


def _topk_body(sc_r, bc_r, st_r, bt_r, o_r, rank_r, cnt_r):
    i = pl.program_id(0)
    j = pl.program_id(1)

    @pl.when(j == 0)
    def _():
        rank_r[...] = jnp.zeros((RK, 1), jnp.float32)
        cnt_r[...] = jnp.zeros((RK, 1), jnp.float32)

    bi = bc_r[...]
    bj = bt_r[...].reshape(1, RK)
    overlap = jnp.logical_and(jnp.max(bj) >= jnp.min(bi),
                              jnp.min(bj) <= jnp.max(bi))

    @pl.when(overlap)
    def _():
        si = sc_r[...]
        sj = st_r[...].reshape(1, RK)
        same = bi == bj
        ii = lax.broadcasted_iota(jnp.int32, (RK, 1), 0) + i * RK
        jj = lax.broadcasted_iota(jnp.int32, (1, RK), 1) + j * RK
        better = jnp.logical_and(
            same, jnp.logical_or(sj > si,
                                 jnp.logical_and(sj == si, jj < ii)))
        rank_r[...] += jnp.sum(better.astype(jnp.float32), axis=1,
                               keepdims=True)
        cnt_r[...] += jnp.sum(same.astype(jnp.float32), axis=1,
                              keepdims=True)

    @pl.when(j == GK - 1)
    def _():
        k = jnp.maximum(1.0, jnp.ceil(0.5 * cnt_r[...]))
        keep = jnp.logical_and(rank_r[...] < k, bi < B)
        o_r[...] = keep.astype(jnp.float32)


def _topk_call(score, batch_col):
    scoreT = score.reshape(GK, 1, RK)
    batchT = batch_col.reshape(GK, 1, RK)
    return pl.pallas_call(
        _topk_body, grid=(GK, GK),
        in_specs=[
            pl.BlockSpec((RK, 1), lambda i, j: (i, 0)),
            pl.BlockSpec((RK, 1), lambda i, j: (i, 0)),
            pl.BlockSpec((1, 1, RK), lambda i, j: (j, 0, 0)),
            pl.BlockSpec((1, 1, RK), lambda i, j: (j, 0, 0)),
        ],
        out_specs=pl.BlockSpec((RK, 1), lambda i, j: (i, 0)),
        out_shape=jax.ShapeDtypeStruct((NP, 1), jnp.float32),
        scratch_shapes=[pltpu.VMEM((RK, 1), jnp.float32),
                        pltpu.VMEM((RK, 1), jnp.float32)],
    )(score, batch_col, scoreT, batchT)


def _readout_body(x1_r, x2_r, x3_r, x4_r, bb_r, kp_r,
                  w1_r, b1_r, w2_r, b2_r, w3_r, b3_r, o_r,
                  s1, s2, s3, s4, m1, m2, m3, m4, cnt, cntm):
    i = pl.program_id(0)

    @pl.when(i == 0)
    def _():
        for s in (s1, s2, s3, s4):
            s[...] = jnp.zeros((B, H), jnp.float32)
        for m in (m1, m2, m3, m4):
            m[...] = jnp.full((B, H), -jnp.inf, jnp.float32)
        cnt[...] = jnp.zeros((B, 1), jnp.float32)
        cntm[...] = jnp.zeros((B, 1), jnp.float32)

    bb = bb_r[...]
    kp = kp_r[...]
    x1 = x1_r[...]
    x2 = x2_r[...]
    x3 = x3_r[...]
    x4 = x4_r[...]
    oh = (bb == lax.broadcasted_iota(jnp.int32, (1, B), 1)).astype(jnp.float32)
    ohm = oh * kp
    dn = (((0,), (0,)), ((), ()))
    s1[...] += lax.dot_general(oh, x1, dn, preferred_element_type=jnp.float32)
    s2[...] += lax.dot_general(oh, x2, dn, preferred_element_type=jnp.float32)
    s3[...] += lax.dot_general(ohm, x3, dn, preferred_element_type=jnp.float32)
    s4[...] += lax.dot_general(ohm, x4, dn, preferred_element_type=jnp.float32)
    ones_c = jnp.ones((R, 1), jnp.float32)
    cnt[...] += lax.dot_general(oh, ones_c, dn,
                                preferred_element_type=jnp.float32)
    cntm[...] += lax.dot_general(ohm, ones_c, dn,
                                 preferred_element_type=jnp.float32)
    neg = jnp.float32(-jnp.inf)
    kpos = kp > 0.0
    for g in range(B):
        mg = bb == g
        mgm = jnp.logical_and(mg, kpos)
        for m_r, xb, msk in ((m1, x1, mg), (m2, x2, mg),
                             (m3, x3, mgm), (m4, x4, mgm)):
            cur = m_r[pl.ds(g, 1), :]
            new = jnp.max(jnp.where(msk, xb, neg), axis=0, keepdims=True)
            m_r[pl.ds(g, 1), :] = jnp.maximum(cur, new)

    @pl.when(i == G - 1)
    def _():
        c = jnp.maximum(cnt[...], 1.0)
        cm = jnp.maximum(cntm[...], 1.0)
        mx = [jnp.where(m[...] == -jnp.inf, 0.0, m[...])
              for m in (m1, m2, m3, m4)]
        jk0 = jax.nn.gelu(jnp.concatenate(
            [s1[...] / c, s2[...] / c, mx[0], mx[1]], axis=1))
        jk1 = jax.nn.gelu(jnp.concatenate(
            [s3[...] / cm, s4[...] / cm, mx[2], mx[3]], axis=1))
        z = jk0 + jk1
        z = jnp.maximum(jnp.dot(z, w1_r[...],
                                preferred_element_type=jnp.float32)
                        + b1_r[...], 0.0)
        z = jnp.maximum(jnp.dot(z, w2_r[...],
                                preferred_element_type=jnp.float32)
                        + b2_r[...], 0.0)
        o_r[...] = jnp.dot(z, w3_r[...],
                           preferred_element_type=jnp.float32) + b3_r[...]


def _readout_call(x1, x2, x3, x4, batch_col, keep, params):
    nc = 10
    return pl.pallas_call(
        _readout_body, grid=(G,),
        in_specs=[
            _rows(H), _rows(H), _rows(H), _rows(H), _rows(1), _rows(1),
            _full((4 * H, H)), _full((1, H)), _full((H, H)), _full((1, H)),
            _full((H, nc)), _full((1, nc)),
        ],
        out_specs=pl.BlockSpec((B, nc), lambda i: (0, 0)),
        out_shape=jax.ShapeDtypeStruct((B, nc), jnp.float32),
        scratch_shapes=[pltpu.VMEM((B, H), jnp.float32)] * 8
        + [pltpu.VMEM((B, 1), jnp.float32)] * 2,
    )(x1, x2, x3, x4, batch_col, keep,
      params["W1"], params["b1"].reshape(1, H),
      params["W2"], params["b2"].reshape(1, H),
      params["W3"], params["b3"].reshape(1, nc))


def kernel(x, params, edge_index, batch):
    src = edge_index[0]
    dst = edge_index[1]
    pad_n = EP - E
    ar = jnp.arange(pad_n, dtype=jnp.int32)
    pad_src = (ar * 37) % N
    pad_dst = N + (ar % (NP - N))
    srcP = jnp.concatenate([src, pad_src]).reshape(NW, NCHUNK, CH)
    dstP = jnp.concatenate([dst, pad_dst]).reshape(NW, NCHUNK, CH)
    x_p = jnp.pad(x, ((0, NP - N), (0, 0)))
    batch_col = jnp.pad(batch, (0, NP - N),
                        constant_values=B).reshape(NP, 1)

    p0a, p0b = params["block0"]
    p1a, p1b = params["block1"]

    deg0 = _deg_pass_ones(srcP, dstP)

    h0, u1, v1 = _pre_call("emb", x_p, p0a,
                           (params["emb_W"], params["emb_b"]))
    acc1 = _edge_pass(v1, srcP, dstP)
    x1 = _post_call(h0, u1, acc1, deg0, p0a)

    u2, v2 = _pre_call("plain", x1, p0b)
    acc2 = _edge_pass(v2, srcP, dstP)
    x2 = _post_call(x1, u2, acc2, deg0, p0b)

    accp = _edge_pass(x2, srcP, dstP)
    score = _score_call(x2, accp, params["pool"])
    keep = _topk_call(score, batch_col)

    deg2 = _deg_pass_gather(keep.reshape(NP), srcP, dstP)

    xk, u3, v3 = _pre_call("pool", x2, p1a, (score, keep))
    acc3 = _edge_pass(v3, srcP, dstP)
    x3 = _post_call(xk, u3, acc3, deg2, p1a)

    u4, v4 = _pre_call("masked", x3, p1b, keep)
    acc4 = _edge_pass(v4, srcP, dstP)
    x4 = _post_call(x3, u4, acc4, deg2, p1b)

    return _readout_call(x1, x2, x3, x4, batch_col, keep, params)


# speedup vs baseline: 14.7026x; 1.2579x over previous
import functools

import jax
import jax.numpy as jnp
from jax import lax
from jax.experimental import pallas as pl
from jax.experimental.pallas import tpu as pltpu
from jax.experimental.pallas import tpu_sc as plsc

N = 10000
NP = 10240
E = 320000
NW = 32
CH = 128
NCHUNK = 80
GC = 16
EP = NW * NCHUNK * CH
STRIPE = NP // 16
H = 128
FFN = 256
B = 16
R = 1024
G = NP // R
RK = 512
GK = NP // RK


@functools.cache
def _make_edge_kernel():
    _mesh = plsc.VectorSubcoreMesh(core_axis_name="c", subcore_axis_name="s")

    @functools.partial(
        pl.kernel, mesh=_mesh,
        out_type=jax.ShapeDtypeStruct((2, NP, H), jnp.float32),
        scratch_types=[
            pltpu.VMEM((GC, CH), jnp.int32),
            pltpu.VMEM((GC, CH), jnp.int32),
            pltpu.VMEM((CH, H), jnp.float32),
            pltpu.VMEM((CH, H), jnp.float32),
            pltpu.VMEM_SHARED((NP, H), jnp.float32),
            pltpu.SemaphoreType.DMA,
            pltpu.SemaphoreType.DMA,
        ],
    )
    def k(table, srcp, dstp, out, src_v, dst_v, buf0, buf1, acc, sem0, sem1):
        c = lax.axis_index("c")
        s = lax.axis_index("s")
        wid = s * 2 + c
        zz = jnp.zeros((16,), jnp.float32)

        def zb(r, _):
            for c8 in range(H // 16):
                buf0[r, pl.ds(c8 * 16, 16)] = zz
            return 0

        lax.fori_loop(0, CH, zb, 0)
        base = s * STRIPE
        for t in range(STRIPE // CH):
            pltpu.sync_copy(buf0, acc.at[pl.ds(base + t * CH, CH)])
        plsc.subcore_barrier()

        for g in range(NCHUNK // GC):
            pltpu.sync_copy(srcp.at[wid, pl.ds(g * GC, GC)], src_v)
            pltpu.sync_copy(dstp.at[wid, pl.ds(g * GC, GC)], dst_v)
            pltpu.async_copy(table.at[src_v.at[0]], buf0, sem0)
            pltpu.async_copy(table.at[src_v.at[1]], buf1, sem1)

            def body(t, _):
                a = 2 * t
                pltpu.make_async_copy(table.at[src_v.at[0]], buf0,
                                      sem0).wait()
                pltpu.sync_copy(buf0, acc.at[dst_v.at[a]], add=True)

                @pl.when(t < GC // 2 - 1)
                def _():
                    pltpu.async_copy(table.at[src_v.at[a + 2]], buf0, sem0)

                pltpu.make_async_copy(table.at[src_v.at[1]], buf1,
                                      sem1).wait()
                pltpu.sync_copy(buf1, acc.at[dst_v.at[a + 1]], add=True)

                @pl.when(t < GC // 2 - 1)
                def _():
                    pltpu.async_copy(table.at[src_v.at[a + 3]], buf1, sem1)

                return 0

            lax.fori_loop(0, GC // 2, body, 0)
        plsc.subcore_barrier()
        pltpu.sync_copy(acc.at[pl.ds(base, STRIPE)],
                        out.at[c, pl.ds(base, STRIPE)])

    return k


@functools.cache
def _make_deg_kernel(gather):
    _mesh = plsc.VectorSubcoreMesh(core_axis_name="c", subcore_axis_name="s")
    scratch = [
        pltpu.VMEM((NCHUNK, CH), jnp.int32),
        pltpu.VMEM((NCHUNK, CH), jnp.int32),
        pltpu.VMEM((CH,), jnp.float32),
        pltpu.VMEM_SHARED((NP,), jnp.float32),
        pltpu.SemaphoreType.DMA,
    ]

    def body_common(c, s, srcp, dstp, out, src_v, dst_v, buf, acc, sem, vals):
        wid = s * 2 + c
        pltpu.sync_copy(srcp.at[wid], src_v)
        pltpu.sync_copy(dstp.at[wid], dst_v)
        zz = jnp.zeros((16,), jnp.float32)

        def zb(i, _):
            buf[pl.ds(i * 16, 16)] = zz
            return 0

        lax.fori_loop(0, CH // 16, zb, 0)
        base = s * STRIPE
        for t in range(STRIPE // CH):
            pltpu.sync_copy(buf, acc.at[pl.ds(base + t * CH, CH)])
        plsc.subcore_barrier()
        if not gather:
            oo = jnp.ones((16,), jnp.float32)

            def ob(i, _):
                buf[pl.ds(i * 16, 16)] = oo
                return 0

            lax.fori_loop(0, CH // 16, ob, 0)

        def body(j, _):
            if gather:
                pltpu.async_copy(vals.at[src_v.at[j]], buf, sem).wait()
            pltpu.sync_copy(buf, acc.at[dst_v.at[j]], add=True)
            return 0

        lax.fori_loop(0, NCHUNK, body, 0)
        plsc.subcore_barrier()
        pltpu.sync_copy(acc.at[pl.ds(base, STRIPE)],
                        out.at[c, pl.ds(base, STRIPE)])

    if gather:
        @functools.partial(
            pl.kernel, mesh=_mesh,
            out_type=jax.ShapeDtypeStruct((2, NP), jnp.float32),
            scratch_types=scratch,
        )
        def kg(vals, srcp, dstp, out, src_v, dst_v, buf, acc, sem):
            body_common(lax.axis_index("c"), lax.axis_index("s"), srcp, dstp,
                        out, src_v, dst_v, buf, acc, sem, vals)

        return kg

    @functools.partial(
        pl.kernel, mesh=_mesh,
        out_type=jax.ShapeDtypeStruct((2, NP), jnp.float32),
        scratch_types=scratch,
    )
    def k1(srcp, dstp, out, src_v, dst_v, buf, acc, sem):
        body_common(lax.axis_index("c"), lax.axis_index("s"), srcp, dstp,
                    out, src_v, dst_v, buf, acc, sem, None)

    return k1


def _edge_pass(table, srcP, dstP):
    return _make_edge_kernel()(table, srcP, dstP)


def _deg_pass_ones(srcP, dstP):
    return _make_deg_kernel(False)(srcP, dstP)


def _deg_pass_gather(vals, srcP, dstP):
    return _make_deg_kernel(True)(vals, srcP, dstP)


def _ln_in(xb, g, b):
    m = jnp.mean(xb, -1, keepdims=True)
    d = xb - m
    v = jnp.mean(d * d, -1, keepdims=True)
    return d / jnp.sqrt(v + 1e-5) * g + b


def _full(shape):
    return pl.BlockSpec(shape, lambda *_: tuple(0 for _ in shape))


def _rows(width):
    return pl.BlockSpec((R, width), lambda i: (i, 0))


def _pre_body(mode, *refs):
    if mode == "emb":
        (x_r, ew_r, eb_r, g1_r, b1_r, wi_r, bi_r, g2_r, b2_r,
         h0_r, u_r, v_r) = refs
    elif mode == "pool":
        (x_r, sc_r, kp_r, g1_r, b1_r, wi_r, bi_r, g2_r, b2_r,
         h0_r, u_r, v_r) = refs
    elif mode == "masked":
        (x_r, kp_r, g1_r, b1_r, wi_r, bi_r, g2_r, b2_r, u_r, v_r) = refs
    else:
        (x_r, g1_r, b1_r, wi_r, bi_r, g2_r, b2_r, u_r, v_r) = refs
    xb = x_r[...]
    if mode == "emb":
        xb = jnp.dot(xb, ew_r[...],
                     preferred_element_type=jnp.float32) + eb_r[...]
        h0_r[...] = xb
    if mode in ("pool", "masked"):
        kp = kp_r[...]
    if mode == "pool":
        xb = xb * jnp.tanh(sc_r[...]) * kp
        h0_r[...] = xb
    h = _ln_in(xb, g1_r[...], b1_r[...])
    h = jnp.dot(h, wi_r[...], preferred_element_type=jnp.float32) + bi_r[...]
    h = jax.nn.gelu(h)
    u = h[:, :H]
    v = _ln_in(h[:, H:], g2_r[...], b2_r[...])
    if mode in ("pool", "masked"):
        v = v * kp
    u_r[...] = u
    v_r[...] = v


def _pre_call(mode, x, p, extra=None):
    fout = jax.ShapeDtypeStruct((NP, H), jnp.float32)
    body = functools.partial(_pre_body, mode)
    in_specs = [_rows(H)]
    args = [x]
    if mode == "emb":
        in_specs += [_full((H, H)), _full((1, H))]
        args += [extra[0], extra[1].reshape(1, H)]
    if mode == "pool":
        in_specs += [_rows(1), _rows(1)]
        args += [extra[0], extra[1]]
    if mode == "masked":
        in_specs += [_rows(1)]
        args += [extra]
    in_specs += [_full((1, H)), _full((1, H)), _full((H, FFN)),
                 _full((1, FFN)), _full((1, H)), _full((1, H))]
    args += [p["ln1_g"].reshape(1, H), p["ln1_b"].reshape(1, H), p["Win"],
             p["bin"].reshape(1, FFN), p["ln2_g"].reshape(1, H),
             p["ln2_b"].reshape(1, H)]
    n_out = 3 if mode in ("emb", "pool") else 2
    return pl.pallas_call(
        body, grid=(G,),
        in_specs=in_specs,
        out_specs=[_rows(H)] * n_out,
        out_shape=[fout] * n_out,
    )(*args)


def _post_body(refs):
    (x_r, u_r, acc_r, deg_r, ws_r, bs_r, wo_r, bo_r, o_r) = refs
    acc = acc_r[0] + acc_r[1]
    dg = deg_r[0] + deg_r[1]
    agg = acc / jnp.maximum(dg, 1.0)
    gate = jnp.dot(agg, ws_r[...],
                   preferred_element_type=jnp.float32) + bs_r[...]
    o_r[...] = x_r[...] + jnp.dot(
        u_r[...] * gate, wo_r[...],
        preferred_element_type=jnp.float32) + bo_r[...]


def _post_call(xin, u, accP, degP, p):
    return pl.pallas_call(
        lambda *refs: _post_body(refs), grid=(G,),
        in_specs=[
            _rows(H), _rows(H),
            pl.BlockSpec((2, R, H), lambda i: (0, i, 0)),
            pl.BlockSpec((2, R, 1), lambda i: (0, i, 0)),
            _full((H, H)), _full((1, H)), _full((H, H)), _full((1, H)),
        ],
        out_specs=_rows(H),
        out_shape=jax.ShapeDtypeStruct((NP, H), jnp.float32),
    )(xin, u, accP, degP.reshape(2, NP, 1), p["Wsgu"],
      p["bsgu"].reshape(1, H), p["Wout"], p["bout"].reshape(1, H))


def _score_body(refs):
    (x_r, acc_r, w1_r, w2_r, b_r, o_r) = refs
    nb = acc_r[0] + acc_r[1]
    o_r[...] = (jnp.dot(x_r[...], w1_r[...],
                        preferred_element_type=jnp.float32)
                + jnp.dot(nb, w2_r[...],
                          preferred_element_type=jnp.float32) + b_r[...])


def _score_call(x, accP, pp):
    return pl.pallas_call(
        lambda *refs: _score_body(refs), grid=(G,),
        in_specs=[
            _rows(H),
            pl.BlockSpec((2, R, H), lambda i: (0, i, 0)),
            _full((H, 1)), _full((H, 1)), _full((1, 1)),
        ],
        out_specs=_rows(1),
        out_shape=jax.ShapeDtypeStruct((NP, 1), jnp.float32),
    )(x, accP, pp["W1"], pp["W2"], pp["b"].reshape(1, 1))


def _topk_body(sc_r, bc_r, st_r, bt_r, o_r, rank_r, cnt_r):
    i = pl.program_id(0)
    j = pl.program_id(1)

    @pl.when(j == 0)
    def _():
        rank_r[...] = jnp.zeros((RK, 1), jnp.float32)
        cnt_r[...] = jnp.zeros((RK, 1), jnp.float32)

    bi = bc_r[...]
    bj = bt_r[...].reshape(1, RK)
    overlap = jnp.logical_and(jnp.max(bj) >= jnp.min(bi),
                              jnp.min(bj) <= jnp.max(bi))

    @pl.when(overlap)
    def _():
        si = sc_r[...]
        sj = st_r[...].reshape(1, RK)
        same = bi == bj
        ii = lax.broadcasted_iota(jnp.int32, (RK, 1), 0) + i * RK
        jj = lax.broadcasted_iota(jnp.int32, (1, RK), 1) + j * RK
        better = jnp.logical_and(
            same, jnp.logical_or(sj > si,
                                 jnp.logical_and(sj == si, jj < ii)))
        rank_r[...] += jnp.sum(better.astype(jnp.float32), axis=1,
                               keepdims=True)
        cnt_r[...] += jnp.sum(same.astype(jnp.float32), axis=1,
                              keepdims=True)

    @pl.when(j == GK - 1)
    def _():
        k = jnp.maximum(1.0, jnp.ceil(0.5 * cnt_r[...]))
        keep = jnp.logical_and(rank_r[...] < k, bi < B)
        o_r[...] = keep.astype(jnp.float32)


def _topk_call(score, batch_col):
    scoreT = score.reshape(GK, 1, RK)
    batchT = batch_col.reshape(GK, 1, RK)
    return pl.pallas_call(
        _topk_body, grid=(GK, GK),
        in_specs=[
            pl.BlockSpec((RK, 1), lambda i, j: (i, 0)),
            pl.BlockSpec((RK, 1), lambda i, j: (i, 0)),
            pl.BlockSpec((1, 1, RK), lambda i, j: (j, 0, 0)),
            pl.BlockSpec((1, 1, RK), lambda i, j: (j, 0, 0)),
        ],
        out_specs=pl.BlockSpec((RK, 1), lambda i, j: (i, 0)),
        out_shape=jax.ShapeDtypeStruct((NP, 1), jnp.float32),
        scratch_shapes=[pltpu.VMEM((RK, 1), jnp.float32),
                        pltpu.VMEM((RK, 1), jnp.float32)],
    )(score, batch_col, scoreT, batchT)


def _readout_body(x1_r, x2_r, x3_r, x4_r, bb_r, kp_r,
                  w1_r, b1_r, w2_r, b2_r, w3_r, b3_r, o_r,
                  s1, s2, s3, s4, m1, m2, m3, m4, cnt, cntm):
    i = pl.program_id(0)

    @pl.when(i == 0)
    def _():
        for s in (s1, s2, s3, s4):
            s[...] = jnp.zeros((B, H), jnp.float32)
        for m in (m1, m2, m3, m4):
            m[...] = jnp.full((B, H), -jnp.inf, jnp.float32)
        cnt[...] = jnp.zeros((B, 1), jnp.float32)
        cntm[...] = jnp.zeros((B, 1), jnp.float32)

    bb = bb_r[...]
    kp = kp_r[...]
    x1 = x1_r[...]
    x2 = x2_r[...]
    x3 = x3_r[...]
    x4 = x4_r[...]
    oh = (bb == lax.broadcasted_iota(jnp.int32, (1, B), 1)).astype(jnp.float32)
    ohm = oh * kp
    dn = (((0,), (0,)), ((), ()))
    s1[...] += lax.dot_general(oh, x1, dn, preferred_element_type=jnp.float32)
    s2[...] += lax.dot_general(oh, x2, dn, preferred_element_type=jnp.float32)
    s3[...] += lax.dot_general(ohm, x3, dn, preferred_element_type=jnp.float32)
    s4[...] += lax.dot_general(ohm, x4, dn, preferred_element_type=jnp.float32)
    ones_c = jnp.ones((R, 1), jnp.float32)
    cnt[...] += lax.dot_general(oh, ones_c, dn,
                                preferred_element_type=jnp.float32)
    cntm[...] += lax.dot_general(ohm, ones_c, dn,
                                 preferred_element_type=jnp.float32)
    neg = jnp.float32(-jnp.inf)
    kpos = kp > 0.0
    for g in range(B):
        mg = bb == g
        mgm = jnp.logical_and(mg, kpos)
        for m_r, xb, msk in ((m1, x1, mg), (m2, x2, mg),
                             (m3, x3, mgm), (m4, x4, mgm)):
            cur = m_r[pl.ds(g, 1), :]
            new = jnp.max(jnp.where(msk, xb, neg), axis=0, keepdims=True)
            m_r[pl.ds(g, 1), :] = jnp.maximum(cur, new)

    @pl.when(i == G - 1)
    def _():
        c = jnp.maximum(cnt[...], 1.0)
        cm = jnp.maximum(cntm[...], 1.0)
        mx = [jnp.where(m[...] == -jnp.inf, 0.0, m[...])
              for m in (m1, m2, m3, m4)]
        jk0 = jax.nn.gelu(jnp.concatenate(
            [s1[...] / c, s2[...] / c, mx[0], mx[1]], axis=1))
        jk1 = jax.nn.gelu(jnp.concatenate(
            [s3[...] / cm, s4[...] / cm, mx[2], mx[3]], axis=1))
        z = jk0 + jk1
        z = jnp.maximum(jnp.dot(z, w1_r[...],
                                preferred_element_type=jnp.float32)
                        + b1_r[...], 0.0)
        z = jnp.maximum(jnp.dot(z, w2_r[...],
                                preferred_element_type=jnp.float32)
                        + b2_r[...], 0.0)
        o_r[...] = jnp.dot(z, w3_r[...],
                           preferred_element_type=jnp.float32) + b3_r[...]


def _readout_call(x1, x2, x3, x4, batch_col, keep, params):
    nc = 10
    return pl.pallas_call(
        _readout_body, grid=(G,),
        in_specs=[
            _rows(H), _rows(H), _rows(H), _rows(H), _rows(1), _rows(1),
            _full((4 * H, H)), _full((1, H)), _full((H, H)), _full((1, H)),
            _full((H, nc)), _full((1, nc)),
        ],
        out_specs=pl.BlockSpec((B, nc), lambda i: (0, 0)),
        out_shape=jax.ShapeDtypeStruct((B, nc), jnp.float32),
        scratch_shapes=[pltpu.VMEM((B, H), jnp.float32)] * 8
        + [pltpu.VMEM((B, 1), jnp.float32)] * 2,
    )(x1, x2, x3, x4, batch_col, keep,
      params["W1"], params["b1"].reshape(1, H),
      params["W2"], params["b2"].reshape(1, H),
      params["W3"], params["b3"].reshape(1, nc))


def kernel(x, params, edge_index, batch):
    src = edge_index[0]
    dst = edge_index[1]
    pad_n = EP - E
    ar = jnp.arange(pad_n, dtype=jnp.int32)
    pad_src = (ar * 37) % N
    pad_dst = N + (ar % (NP - N))
    srcP = jnp.concatenate([src, pad_src]).reshape(NW, NCHUNK, CH)
    dstP = jnp.concatenate([dst, pad_dst]).reshape(NW, NCHUNK, CH)
    x_p = jnp.pad(x, ((0, NP - N), (0, 0)))
    batch_col = jnp.pad(batch, (0, NP - N),
                        constant_values=B).reshape(NP, 1)

    p0a, p0b = params["block0"]
    p1a, p1b = params["block1"]

    deg0 = _deg_pass_ones(srcP, dstP)

    h0, u1, v1 = _pre_call("emb", x_p, p0a,
                           (params["emb_W"], params["emb_b"]))
    acc1 = _edge_pass(v1, srcP, dstP)
    x1 = _post_call(h0, u1, acc1, deg0, p0a)

    u2, v2 = _pre_call("plain", x1, p0b)
    acc2 = _edge_pass(v2, srcP, dstP)
    x2 = _post_call(x1, u2, acc2, deg0, p0b)

    accp = _edge_pass(x2, srcP, dstP)
    score = _score_call(x2, accp, params["pool"])
    keep = _topk_call(score, batch_col)

    deg2 = _deg_pass_gather(keep.reshape(NP), srcP, dstP)

    xk, u3, v3 = _pre_call("pool", x2, p1a, (score, keep))
    acc3 = _edge_pass(v3, srcP, dstP)
    x3 = _post_call(xk, u3, acc3, deg2, p1a)

    u4, v4 = _pre_call("masked", x3, p1b, keep)
    acc4 = _edge_pass(v4, srcP, dstP)
    x4 = _post_call(x3, u4, acc4, deg2, p1b)

    return _readout_call(x1, x2, x3, x4, batch_col, keep, params)
